# baseline (speedup 1.0000x reference)
import functools

import jax
import jax.numpy as jnp
from jax import lax
from jax.experimental import pallas as pl
from jax.experimental.pallas import tpu as pltpu
from jax.experimental.pallas import tpu_sc as plsc

T, H, I, E, A, R, K = 2048, 768, 1536, 8, 4, 16, 2
BLK = 128
NB = (T * K) // BLK + E - 1
NB = NB + (-NB) % 2
NPAD = NB * BLK

NW = 32
G_PER_W = NPAD // NW
G_CHUNK = 80
C_PER_W = T // NW


def _wid():
    return lax.axis_index("s") * 2 + lax.axis_index("c")


def _sc_dist_body(x_hbm, p0_hbm, p1_hbm, sx_hbm, i0_v, i1_v, x_v, sem):
    base = _wid() * C_PER_W
    pltpu.sync_copy(p0_hbm.at[pl.ds(base, C_PER_W)], i0_v)
    pltpu.sync_copy(p1_hbm.at[pl.ds(base, C_PER_W)], i1_v)
    pltpu.sync_copy(x_hbm.at[pl.ds(base, C_PER_W)], x_v)
    cps = [pltpu.async_copy(x_v, sx_hbm.at[i0_v], sem),
           pltpu.async_copy(x_v, sx_hbm.at[i1_v], sem)]
    for c in cps:
        c.wait()


def _sc_combine_body(so_hbm, p0_hbm, p1_hbm, w0_hbm, w1_hbm, out_hbm,
                     i0_v, i1_v, a_v, b_v, w0_v, w1_v, sem):
    base = _wid() * C_PER_W
    pltpu.sync_copy(p0_hbm.at[pl.ds(base, C_PER_W)], i0_v)
    pltpu.sync_copy(p1_hbm.at[pl.ds(base, C_PER_W)], i1_v)
    pltpu.sync_copy(w0_hbm.at[pl.ds(base, C_PER_W)], w0_v)
    pltpu.sync_copy(w1_hbm.at[pl.ds(base, C_PER_W)], w1_v)
    cp_a = pltpu.async_copy(so_hbm.at[i0_v], a_v, sem)
    cp_b = pltpu.async_copy(so_hbm.at[i1_v], b_v, sem)
    cp_a.wait()
    cp_b.wait()

    def row(r, _):
        wa = w0_v[r, pl.ds(0, 16)]
        wb = w1_v[r, pl.ds(0, 16)]
        for c in range(H // 16):
            s = pl.ds(c * 16, 16)
            a_v[r, s] = a_v[r, s] * wa + b_v[r, s] * wb
        return 0

    lax.fori_loop(0, C_PER_W, row, 0)
    pltpu.sync_copy(a_v, out_hbm.at[pl.ds(base, C_PER_W)])


def _sc_distribute(x, p0, p1):
    mesh = plsc.VectorSubcoreMesh(core_axis_name="c", subcore_axis_name="s")
    f = functools.partial(
        pl.kernel, mesh=mesh,
        out_type=jax.ShapeDtypeStruct((NPAD, H), jnp.float32),
        scratch_types=[
            pltpu.VMEM((C_PER_W,), jnp.int32),
            pltpu.VMEM((C_PER_W,), jnp.int32),
            pltpu.VMEM((C_PER_W, H), jnp.float32),
            pltpu.SemaphoreType.DMA,
        ])(_sc_dist_body)
    return f(x, p0, p1)


def _sc_combine(sorted_out, p0, p1, w0r, w1r):
    mesh = plsc.VectorSubcoreMesh(core_axis_name="c", subcore_axis_name="s")
    f = functools.partial(
        pl.kernel, mesh=mesh,
        out_type=jax.ShapeDtypeStruct((T, H), jnp.float32),
        scratch_types=[
            pltpu.VMEM((C_PER_W,), jnp.int32),
            pltpu.VMEM((C_PER_W,), jnp.int32),
            pltpu.VMEM((C_PER_W, H), jnp.float32),
            pltpu.VMEM((C_PER_W, H), jnp.float32),
            pltpu.VMEM((C_PER_W, 16), jnp.float32),
            pltpu.VMEM((C_PER_W, 16), jnp.float32),
            pltpu.SemaphoreType.DMA,
        ])(_sc_combine_body)
    return f(sorted_out, p0, p1, w0r, w1r)


def _tc_body(be_ref, act_ref, x_ref, wgu_ref, wd_ref, ga_ref, gb_ref,
             ua_ref, ub_ref, da_ref, db_ref, out_ref, gate_s):
    del be_ref
    b = pl.program_id(0)
    j = pl.program_id(1)
    active = act_ref[b] == 1

    @pl.when(jnp.logical_and(active, j == 0))
    def _gate_phase():
        xb = x_ref[...]
        g = jnp.dot(xb, wgu_ref[0], preferred_element_type=jnp.float32)
        lg = jnp.dot(jnp.dot(xb, ga_ref[0], preferred_element_type=jnp.float32),
                     gb_ref[0], preferred_element_type=jnp.float32)
        gate_s[...] = g + lg

    @pl.when(jnp.logical_and(active, j == 1))
    def _up_down_phase():
        xb = x_ref[...]
        u = jnp.dot(xb, wgu_ref[0], preferred_element_type=jnp.float32)
        lu = jnp.dot(jnp.dot(xb, ua_ref[0], preferred_element_type=jnp.float32),
                     ub_ref[0], preferred_element_type=jnp.float32)
        up = u + lu
        gate = gate_s[...]
        act = gate * jax.nn.sigmoid(gate) * up
        dn = jnp.dot(act, wd_ref[0], preferred_element_type=jnp.float32)
        ld = jnp.dot(jnp.dot(act, da_ref[0], preferred_element_type=jnp.float32),
                     db_ref[0], preferred_element_type=jnp.float32)
        out_ref[...] = dn + ld

    @pl.when(jnp.logical_and(jnp.logical_not(active), j == 1))
    def _pad():
        out_ref[...] = jnp.zeros_like(out_ref)


def kernel(hidden_states, topk_ids, topk_weights, gate_a, gate_b, up_a, up_b,
           down_a, down_b, weight_indices, seq_lens, lora_ranks, scalings,
           base_gate_up_weight, base_down_weight):
    del seq_lens, lora_ranks
    x = hidden_states.astype(jnp.float32)
    adapter = weight_indices[0]
    scaling = scalings[adapter].astype(jnp.float32)
    ga = jnp.transpose(gate_a[adapter], (0, 2, 1)).astype(jnp.float32)
    gb = jnp.transpose(gate_b[adapter], (0, 2, 1)).astype(jnp.float32) * scaling
    ua = jnp.transpose(up_a[adapter], (0, 2, 1)).astype(jnp.float32)
    ub = jnp.transpose(up_b[adapter], (0, 2, 1)).astype(jnp.float32) * scaling
    da = jnp.transpose(down_a[adapter], (0, 2, 1)).astype(jnp.float32)
    db = jnp.transpose(down_b[adapter], (0, 2, 1)).astype(jnp.float32) * scaling

    flat_ids = topk_ids.reshape(-1)
    flat_w = topk_weights.reshape(-1).astype(jnp.float32)
    oh = (flat_ids[:, None] == jnp.arange(E, dtype=jnp.int32)[None, :])
    g = jnp.sum(oh, axis=0, dtype=jnp.int32)
    blocks_e = (g + BLK - 1) // BLK
    pad_start = jnp.concatenate([jnp.zeros((1,), jnp.int32),
                                 jnp.cumsum(blocks_e)[:-1].astype(jnp.int32)]) * BLK
    rank = jnp.take_along_axis(jnp.cumsum(oh, axis=0, dtype=jnp.int32) - 1,
                               flat_ids[:, None].astype(jnp.int32), 1)[:, 0]
    dest = pad_start[flat_ids] + rank
    cumb = jnp.cumsum(blocks_e)
    block_expert = jnp.minimum(
        jnp.sum(jnp.arange(NB, dtype=jnp.int32)[:, None] >= cumb[None, :],
                axis=1), E - 1).astype(jnp.int32)
    block_active = (jnp.arange(NB, dtype=jnp.int32) < cumb[-1]).astype(jnp.int32)
    pos = dest.reshape(T, K)
    p0, p1 = pos[:, 0], pos[:, 1]
    w0r = jnp.broadcast_to(topk_weights[:, 0:1].astype(jnp.float32), (T, 16))
    w1r = jnp.broadcast_to(topk_weights[:, 1:2].astype(jnp.float32), (T, 16))

    sorted_x = _sc_distribute(x, p0, p1)

    grid_spec = pltpu.PrefetchScalarGridSpec(
        num_scalar_prefetch=2,
        grid=(NB, 2),
        in_specs=[
            pl.BlockSpec((BLK, H), lambda b, j, be, ac: (b, 0)),
            pl.BlockSpec((1, H, I), lambda b, j, be, ac: (be[b], 0, j)),
            pl.BlockSpec((1, I, H), lambda b, j, be, ac: (be[b], 0, 0)),
            pl.BlockSpec((1, H, R), lambda b, j, be, ac: (be[b], 0, 0)),
            pl.BlockSpec((1, R, I), lambda b, j, be, ac: (be[b], 0, 0)),
            pl.BlockSpec((1, H, R), lambda b, j, be, ac: (be[b], 0, 0)),
            pl.BlockSpec((1, R, I), lambda b, j, be, ac: (be[b], 0, 0)),
            pl.BlockSpec((1, I, R), lambda b, j, be, ac: (be[b], 0, 0)),
            pl.BlockSpec((1, R, H), lambda b, j, be, ac: (be[b], 0, 0)),
        ],
        out_specs=pl.BlockSpec((BLK, H), lambda b, j, be, ac: (b, 0)),
        scratch_shapes=[pltpu.VMEM((BLK, I), jnp.float32)],
    )
    sorted_out = pl.pallas_call(
        _tc_body,
        grid_spec=grid_spec,
        out_shape=jax.ShapeDtypeStruct((NPAD, H), jnp.float32),
        compiler_params=pltpu.CompilerParams(
            vmem_limit_bytes=100 * 1024 * 1024),
    )(block_expert, block_active, sorted_x,
      base_gate_up_weight, base_down_weight, ga, gb, ua, ub, da, db)

    out = _sc_combine(sorted_out, p0, p1, w0r, w1r)
    return out.astype(hidden_states.dtype)


# speedup vs baseline: 1.2080x; 1.2080x over previous
import functools

import jax
import jax.numpy as jnp
from jax import lax
from jax.experimental import pallas as pl
from jax.experimental.pallas import tpu as pltpu
from jax.experimental.pallas import tpu_sc as plsc

T, H, I, E, A, R, K = 2048, 768, 1536, 8, 4, 16, 2
BLK = 128
NB = (T * K) // BLK + E - 1
NB = NB + (-NB) % 2
NPAD = NB * BLK

NW = 32
G_PER_W = NPAD // NW
G_CHUNK = 80
C_PER_W = T // NW


def _wid():
    return lax.axis_index("s") * 2 + lax.axis_index("c")


def _sc_dist_body(x_hbm, p0_hbm, p1_hbm, sx_hbm, i0_v, i1_v, x_v, sem):
    base = _wid() * C_PER_W
    pltpu.sync_copy(p0_hbm.at[pl.ds(base, C_PER_W)], i0_v)
    pltpu.sync_copy(p1_hbm.at[pl.ds(base, C_PER_W)], i1_v)
    pltpu.sync_copy(x_hbm.at[pl.ds(base, C_PER_W)], x_v)
    cps = [pltpu.async_copy(x_v, sx_hbm.at[i0_v], sem),
           pltpu.async_copy(x_v, sx_hbm.at[i1_v], sem)]
    for c in cps:
        c.wait()


def _sc_combine_body(so_hbm, p0_hbm, p1_hbm, w0_hbm, w1_hbm, out_hbm,
                     i0_v, i1_v, a_v, b_v, w0_v, w1_v, sem):
    base = _wid() * C_PER_W
    pltpu.sync_copy(p0_hbm.at[pl.ds(base, C_PER_W)], i0_v)
    pltpu.sync_copy(p1_hbm.at[pl.ds(base, C_PER_W)], i1_v)
    pltpu.sync_copy(w0_hbm.at[pl.ds(base, C_PER_W)], w0_v)
    pltpu.sync_copy(w1_hbm.at[pl.ds(base, C_PER_W)], w1_v)
    cp_a = pltpu.async_copy(so_hbm.at[i0_v], a_v, sem)
    cp_b = pltpu.async_copy(so_hbm.at[i1_v], b_v, sem)
    cp_a.wait()
    cp_b.wait()

    def row(r, _):
        wa = w0_v[r, pl.ds(0, 16)]
        wb = w1_v[r, pl.ds(0, 16)]
        for c in range(H // 16):
            s = pl.ds(c * 16, 16)
            a_v[r, s] = a_v[r, s] * wa + b_v[r, s] * wb
        return 0

    lax.fori_loop(0, C_PER_W, row, 0)
    pltpu.sync_copy(a_v, out_hbm.at[pl.ds(base, C_PER_W)])


def _sc_distribute(x, p0, p1):
    mesh = plsc.VectorSubcoreMesh(core_axis_name="c", subcore_axis_name="s")
    f = functools.partial(
        pl.kernel, mesh=mesh,
        out_type=jax.ShapeDtypeStruct((NPAD, H), jnp.float32),
        scratch_types=[
            pltpu.VMEM((C_PER_W,), jnp.int32),
            pltpu.VMEM((C_PER_W,), jnp.int32),
            pltpu.VMEM((C_PER_W, H), jnp.float32),
            pltpu.SemaphoreType.DMA,
        ])(_sc_dist_body)
    return f(x, p0, p1)


def _sc_combine(sorted_out, p0, p1, w0r, w1r):
    mesh = plsc.VectorSubcoreMesh(core_axis_name="c", subcore_axis_name="s")
    f = functools.partial(
        pl.kernel, mesh=mesh,
        out_type=jax.ShapeDtypeStruct((T, H), jnp.float32),
        scratch_types=[
            pltpu.VMEM((C_PER_W,), jnp.int32),
            pltpu.VMEM((C_PER_W,), jnp.int32),
            pltpu.VMEM((C_PER_W, H), jnp.float32),
            pltpu.VMEM((C_PER_W, H), jnp.float32),
            pltpu.VMEM((C_PER_W, 16), jnp.float32),
            pltpu.VMEM((C_PER_W, 16), jnp.float32),
            pltpu.SemaphoreType.DMA,
        ])(_sc_combine_body)
    return f(sorted_out, p0, p1, w0r, w1r)


_NT = (((1,), (1,)), ((), ()))


def _tc_body(be_ref, act_ref, x_ref, wgu_ref, wd_ref, acat_ref, gb_ref,
             ub_ref, da_ref, db_ref, out_ref):
    del be_ref
    b = pl.program_id(0)

    @pl.when(act_ref[b] == 1)
    def _compute():
        xb = x_ref[...]
        gu = jnp.dot(xb, wgu_ref[0], preferred_element_type=jnp.float32)
        xa = lax.dot_general(xb, acat_ref[0], _NT,
                             preferred_element_type=jnp.float32)
        lg = lax.dot_general(xa[:, :R], gb_ref[0], _NT,
                             preferred_element_type=jnp.float32)
        lu = lax.dot_general(xa[:, R:], ub_ref[0], _NT,
                             preferred_element_type=jnp.float32)
        gate = gu[:, :I] + lg
        up = gu[:, I:] + lu
        act = gate * jax.nn.sigmoid(gate) * up
        dn = jnp.dot(act, wd_ref[0], preferred_element_type=jnp.float32)
        x2 = lax.dot_general(act, da_ref[0], _NT,
                             preferred_element_type=jnp.float32)
        ld = lax.dot_general(x2, db_ref[0], _NT,
                             preferred_element_type=jnp.float32)
        out_ref[...] = dn + ld

    @pl.when(act_ref[b] == 0)
    def _pad():
        out_ref[...] = jnp.zeros_like(out_ref)


def kernel(hidden_states, topk_ids, topk_weights, gate_a, gate_b, up_a, up_b,
           down_a, down_b, weight_indices, seq_lens, lora_ranks, scalings,
           base_gate_up_weight, base_down_weight):
    del seq_lens, lora_ranks
    x = hidden_states.astype(jnp.float32)
    adapter = weight_indices[0]
    scaling = scalings[adapter].astype(jnp.float32)
    acat = jnp.concatenate([gate_a[adapter], up_a[adapter]],
                           axis=1).astype(jnp.float32) * scaling
    gb = gate_b[adapter].astype(jnp.float32)
    ub = up_b[adapter].astype(jnp.float32)
    da = down_a[adapter].astype(jnp.float32) * scaling
    db = down_b[adapter].astype(jnp.float32)

    flat_ids = topk_ids.reshape(-1)
    flat_w = topk_weights.reshape(-1).astype(jnp.float32)
    oh = (flat_ids[:, None] == jnp.arange(E, dtype=jnp.int32)[None, :])
    g = jnp.sum(oh, axis=0, dtype=jnp.int32)
    blocks_e = (g + BLK - 1) // BLK
    pad_start = jnp.concatenate([jnp.zeros((1,), jnp.int32),
                                 jnp.cumsum(blocks_e)[:-1].astype(jnp.int32)]) * BLK
    rank = jnp.take_along_axis(jnp.cumsum(oh, axis=0, dtype=jnp.int32) - 1,
                               flat_ids[:, None].astype(jnp.int32), 1)[:, 0]
    dest = pad_start[flat_ids] + rank
    cumb = jnp.cumsum(blocks_e)
    block_expert = jnp.minimum(
        jnp.sum(jnp.arange(NB, dtype=jnp.int32)[:, None] >= cumb[None, :],
                axis=1), E - 1).astype(jnp.int32)
    block_active = (jnp.arange(NB, dtype=jnp.int32) < cumb[-1]).astype(jnp.int32)
    pos = dest.reshape(T, K)
    p0, p1 = pos[:, 0], pos[:, 1]
    w0r = jnp.broadcast_to(topk_weights[:, 0:1].astype(jnp.float32), (T, 16))
    w1r = jnp.broadcast_to(topk_weights[:, 1:2].astype(jnp.float32), (T, 16))

    sorted_x = _sc_distribute(x, p0, p1)

    grid_spec = pltpu.PrefetchScalarGridSpec(
        num_scalar_prefetch=2,
        grid=(NB,),
        in_specs=[
            pl.BlockSpec((BLK, H), lambda b, be, ac: (b, 0)),
            pl.BlockSpec((1, H, 2 * I), lambda b, be, ac: (be[b], 0, 0)),
            pl.BlockSpec((1, I, H), lambda b, be, ac: (be[b], 0, 0)),
            pl.BlockSpec((1, 2 * R, H), lambda b, be, ac: (be[b], 0, 0)),
            pl.BlockSpec((1, I, R), lambda b, be, ac: (be[b], 0, 0)),
            pl.BlockSpec((1, I, R), lambda b, be, ac: (be[b], 0, 0)),
            pl.BlockSpec((1, R, I), lambda b, be, ac: (be[b], 0, 0)),
            pl.BlockSpec((1, H, R), lambda b, be, ac: (be[b], 0, 0)),
        ],
        out_specs=pl.BlockSpec((BLK, H), lambda b, be, ac: (b, 0)),
    )
    sorted_out = pl.pallas_call(
        _tc_body,
        grid_spec=grid_spec,
        out_shape=jax.ShapeDtypeStruct((NPAD, H), jnp.float32),
        compiler_params=pltpu.CompilerParams(
            vmem_limit_bytes=100 * 1024 * 1024),
    )(block_expert, block_active, sorted_x,
      base_gate_up_weight, base_down_weight, acat, gb, ub, da, db)

    out = _sc_combine(sorted_out, p0, p1, w0r, w1r)
    return out.astype(hidden_states.dtype)


# speedup vs baseline: 1.2791x; 1.0589x over previous
import functools

import jax
import jax.numpy as jnp
from jax import lax
from jax.experimental import pallas as pl
from jax.experimental.pallas import tpu as pltpu
from jax.experimental.pallas import tpu_sc as plsc

T, H, I, E, A, R, K = 2048, 768, 1536, 8, 4, 16, 2
BLK = 128
NB = (T * K) // BLK + E - 1
NB = NB + (-NB) % 2
NPAD = NB * BLK

NW = 32
G_PER_W = NPAD // NW
G_CHUNK = 80
C_PER_W = T // NW


def _wid():
    return lax.axis_index("s") * 2 + lax.axis_index("c")


def _sc_dist_body(x_hbm, p0_hbm, p1_hbm, sx_hbm, i0_v, i1_v, x_v, sem):
    base = _wid() * C_PER_W
    pltpu.sync_copy(p0_hbm.at[pl.ds(base, C_PER_W)], i0_v)
    pltpu.sync_copy(p1_hbm.at[pl.ds(base, C_PER_W)], i1_v)
    pltpu.sync_copy(x_hbm.at[pl.ds(base, C_PER_W)], x_v)
    cps = [pltpu.async_copy(x_v, sx_hbm.at[i0_v], sem),
           pltpu.async_copy(x_v, sx_hbm.at[i1_v], sem)]
    for c in cps:
        c.wait()


def _sc_combine_body(so_hbm, p0_hbm, p1_hbm, w0_hbm, w1_hbm, out_hbm,
                     i0_v, i1_v, a_v, b_v, w0_v, w1_v, sem):
    base = _wid() * C_PER_W
    pltpu.sync_copy(p0_hbm.at[pl.ds(base, C_PER_W)], i0_v)
    pltpu.sync_copy(p1_hbm.at[pl.ds(base, C_PER_W)], i1_v)
    pltpu.sync_copy(w0_hbm.at[pl.ds(base, C_PER_W)], w0_v)
    pltpu.sync_copy(w1_hbm.at[pl.ds(base, C_PER_W)], w1_v)
    cp_a = pltpu.async_copy(so_hbm.at[i0_v], a_v, sem)
    cp_b = pltpu.async_copy(so_hbm.at[i1_v], b_v, sem)
    cp_a.wait()
    cp_b.wait()

    def row(r, _):
        wa = w0_v[r, pl.ds(0, 16)]
        wb = w1_v[r, pl.ds(0, 16)]
        for c in range(H // 16):
            s = pl.ds(c * 16, 16)
            a_v[r, s] = a_v[r, s] * wa + b_v[r, s] * wb
        return 0

    lax.fori_loop(0, C_PER_W, row, 0)
    pltpu.sync_copy(a_v, out_hbm.at[pl.ds(base, C_PER_W)])


def _sc_distribute(x, p0, p1):
    mesh = plsc.VectorSubcoreMesh(core_axis_name="c", subcore_axis_name="s")
    f = functools.partial(
        pl.kernel, mesh=mesh,
        out_type=jax.ShapeDtypeStruct((NPAD, H), jnp.float32),
        scratch_types=[
            pltpu.VMEM((C_PER_W,), jnp.int32),
            pltpu.VMEM((C_PER_W,), jnp.int32),
            pltpu.VMEM((C_PER_W, H), jnp.float32),
            pltpu.SemaphoreType.DMA,
        ])(_sc_dist_body)
    return f(x, p0, p1)


def _sc_combine(sorted_out, p0, p1, w0r, w1r):
    mesh = plsc.VectorSubcoreMesh(core_axis_name="c", subcore_axis_name="s")
    f = functools.partial(
        pl.kernel, mesh=mesh,
        out_type=jax.ShapeDtypeStruct((T, H), jnp.float32),
        scratch_types=[
            pltpu.VMEM((C_PER_W,), jnp.int32),
            pltpu.VMEM((C_PER_W,), jnp.int32),
            pltpu.VMEM((C_PER_W, H), jnp.float32),
            pltpu.VMEM((C_PER_W, H), jnp.float32),
            pltpu.VMEM((C_PER_W, 16), jnp.float32),
            pltpu.VMEM((C_PER_W, 16), jnp.float32),
            pltpu.SemaphoreType.DMA,
        ])(_sc_combine_body)
    return f(sorted_out, p0, p1, w0r, w1r)


def _tc_body(be_ref, act_ref, x_ref, wgu_ref, wd_ref, acat_ref, gb_ref,
             ub_ref, da_ref, db_ref, out_ref):
    del be_ref
    b = pl.program_id(0)

    @pl.when(act_ref[b] == 1)
    def _compute():
        xb = x_ref[...]
        gu = jnp.dot(xb, wgu_ref[0], preferred_element_type=jnp.float32)
        xa = jnp.dot(xb, acat_ref[0], preferred_element_type=jnp.float32)
        lg = jnp.dot(xa[:, :R], gb_ref[0], preferred_element_type=jnp.float32)
        lu = jnp.dot(xa[:, R:], ub_ref[0], preferred_element_type=jnp.float32)
        gate = gu[:, :I] + lg
        up = gu[:, I:] + lu
        act = gate * jax.nn.sigmoid(gate) * up
        dn = jnp.dot(act, wd_ref[0], preferred_element_type=jnp.float32)
        x2 = jnp.dot(act, da_ref[0], preferred_element_type=jnp.float32)
        ld = jnp.dot(x2, db_ref[0], preferred_element_type=jnp.float32)
        out_ref[...] = dn + ld

    @pl.when(act_ref[b] == 0)
    def _pad():
        out_ref[...] = jnp.zeros_like(out_ref)


def kernel(hidden_states, topk_ids, topk_weights, gate_a, gate_b, up_a, up_b,
           down_a, down_b, weight_indices, seq_lens, lora_ranks, scalings,
           base_gate_up_weight, base_down_weight):
    del seq_lens, lora_ranks
    x = hidden_states.astype(jnp.float32)
    adapter = weight_indices[0]
    scaling = scalings[adapter].astype(jnp.float32)
    acat = jnp.transpose(
        jnp.concatenate([gate_a[adapter], up_a[adapter]], axis=1),
        (0, 2, 1)).astype(jnp.float32)
    gb = jnp.transpose(gate_b[adapter], (0, 2, 1)).astype(jnp.float32) * scaling
    ub = jnp.transpose(up_b[adapter], (0, 2, 1)).astype(jnp.float32) * scaling
    da = jnp.transpose(down_a[adapter], (0, 2, 1)).astype(jnp.float32)
    db = jnp.transpose(down_b[adapter], (0, 2, 1)).astype(jnp.float32) * scaling

    flat_ids = topk_ids.reshape(-1)
    flat_w = topk_weights.reshape(-1).astype(jnp.float32)
    oh = (flat_ids[:, None] == jnp.arange(E, dtype=jnp.int32)[None, :])
    g = jnp.sum(oh, axis=0, dtype=jnp.int32)
    blocks_e = (g + BLK - 1) // BLK
    pad_start = jnp.concatenate([jnp.zeros((1,), jnp.int32),
                                 jnp.cumsum(blocks_e)[:-1].astype(jnp.int32)]) * BLK
    rank = jnp.take_along_axis(jnp.cumsum(oh, axis=0, dtype=jnp.int32) - 1,
                               flat_ids[:, None].astype(jnp.int32), 1)[:, 0]
    dest = pad_start[flat_ids] + rank
    cumb = jnp.cumsum(blocks_e)
    block_expert = jnp.minimum(
        jnp.sum(jnp.arange(NB, dtype=jnp.int32)[:, None] >= cumb[None, :],
                axis=1), E - 1).astype(jnp.int32)
    block_active = (jnp.arange(NB, dtype=jnp.int32) < cumb[-1]).astype(jnp.int32)
    pos = dest.reshape(T, K)
    p0, p1 = pos[:, 0], pos[:, 1]
    w0r = jnp.broadcast_to(topk_weights[:, 0:1].astype(jnp.float32), (T, 16))
    w1r = jnp.broadcast_to(topk_weights[:, 1:2].astype(jnp.float32), (T, 16))

    sorted_x = _sc_distribute(x, p0, p1)

    grid_spec = pltpu.PrefetchScalarGridSpec(
        num_scalar_prefetch=2,
        grid=(NB,),
        in_specs=[
            pl.BlockSpec((BLK, H), lambda b, be, ac: (b, 0)),
            pl.BlockSpec((1, H, 2 * I), lambda b, be, ac: (be[b], 0, 0)),
            pl.BlockSpec((1, I, H), lambda b, be, ac: (be[b], 0, 0)),
            pl.BlockSpec((1, H, 2 * R), lambda b, be, ac: (be[b], 0, 0)),
            pl.BlockSpec((1, R, I), lambda b, be, ac: (be[b], 0, 0)),
            pl.BlockSpec((1, R, I), lambda b, be, ac: (be[b], 0, 0)),
            pl.BlockSpec((1, I, R), lambda b, be, ac: (be[b], 0, 0)),
            pl.BlockSpec((1, R, H), lambda b, be, ac: (be[b], 0, 0)),
        ],
        out_specs=pl.BlockSpec((BLK, H), lambda b, be, ac: (b, 0)),
    )
    sorted_out = pl.pallas_call(
        _tc_body,
        grid_spec=grid_spec,
        out_shape=jax.ShapeDtypeStruct((NPAD, H), jnp.float32),
        compiler_params=pltpu.CompilerParams(
            vmem_limit_bytes=100 * 1024 * 1024),
    )(block_expert, block_active, sorted_x,
      base_gate_up_weight, base_down_weight, acat, gb, ub, da, db)

    out = _sc_combine(sorted_out, p0, p1, w0r, w1r)
    return out.astype(hidden_states.dtype)


# speedup vs baseline: 1.3088x; 1.0232x over previous
import functools

import jax
import jax.numpy as jnp
from jax import lax
from jax.experimental import pallas as pl
from jax.experimental.pallas import tpu as pltpu
from jax.experimental.pallas import tpu_sc as plsc

T, H, I, E, A, R, K = 2048, 768, 1536, 8, 4, 16, 2
BLK = 128
NB = (T * K) // BLK + E - 1
NB = NB + (-NB) % 2
NPAD = NB * BLK

NW = 32
G_PER_W = NPAD // NW
G_CHUNK = 80
C_PER_W = T // NW


def _wid():
    return lax.axis_index("s") * 2 + lax.axis_index("c")


def _sc_dist_body(x_hbm, p0_hbm, p1_hbm, sx_hbm, i0_v, i1_v, x_v, lsem, sem):
    base = _wid() * C_PER_W
    lds = [pltpu.async_copy(p0_hbm.at[pl.ds(base, C_PER_W)], i0_v, lsem),
           pltpu.async_copy(p1_hbm.at[pl.ds(base, C_PER_W)], i1_v, lsem),
           pltpu.async_copy(x_hbm.at[pl.ds(base, C_PER_W)], x_v, lsem)]
    for c in lds:
        c.wait()
    cps = [pltpu.async_copy(x_v, sx_hbm.at[i0_v], sem),
           pltpu.async_copy(x_v, sx_hbm.at[i1_v], sem)]
    for c in cps:
        c.wait()


def _sc_combine_body(so_hbm, p0_hbm, p1_hbm, w0_hbm, w1_hbm, out_hbm,
                     i0_v, i1_v, a_v, b_v, w0_v, w1_v, gsem, ssem):
    base = _wid() * C_PER_W
    lds = [pltpu.async_copy(p0_hbm.at[pl.ds(base, C_PER_W)], i0_v, ssem),
           pltpu.async_copy(p1_hbm.at[pl.ds(base, C_PER_W)], i1_v, ssem),
           pltpu.async_copy(w0_hbm.at[pl.ds(base, C_PER_W)], w0_v, ssem),
           pltpu.async_copy(w1_hbm.at[pl.ds(base, C_PER_W)], w1_v, ssem)]
    for c in lds:
        c.wait()
    HALF = C_PER_W // 2
    gs = []
    for c in range(2):
        s = pl.ds(c * HALF, HALF)
        gs.append(pltpu.async_copy(so_hbm.at[i0_v.at[s]], a_v.at[s], gsem))
        gs.append(pltpu.async_copy(so_hbm.at[i1_v.at[s]], b_v.at[s], gsem))

    def row(r, _):
        wa = w0_v[r, pl.ds(0, 16)]
        wb = w1_v[r, pl.ds(0, 16)]
        for c in range(H // 16):
            s = pl.ds(c * 16, 16)
            a_v[r, s] = a_v[r, s] * wa + b_v[r, s] * wb
        return 0

    stores = []
    for c in range(2):
        gs[2 * c].wait()
        gs[2 * c + 1].wait()
        lax.fori_loop(c * HALF, (c + 1) * HALF, row, 0)
        stores.append(pltpu.async_copy(
            a_v.at[pl.ds(c * HALF, HALF)],
            out_hbm.at[pl.ds(base + c * HALF, HALF)], ssem))
    for st in stores:
        st.wait()


def _sc_distribute(x, p0, p1):
    mesh = plsc.VectorSubcoreMesh(core_axis_name="c", subcore_axis_name="s")
    f = functools.partial(
        pl.kernel, mesh=mesh,
        out_type=jax.ShapeDtypeStruct((NPAD, H), jnp.float32),
        scratch_types=[
            pltpu.VMEM((C_PER_W,), jnp.int32),
            pltpu.VMEM((C_PER_W,), jnp.int32),
            pltpu.VMEM((C_PER_W, H), jnp.float32),
            pltpu.SemaphoreType.DMA,
            pltpu.SemaphoreType.DMA,
        ])(_sc_dist_body)
    return f(x, p0, p1)


def _sc_combine(sorted_out, p0, p1, w0r, w1r):
    mesh = plsc.VectorSubcoreMesh(core_axis_name="c", subcore_axis_name="s")
    f = functools.partial(
        pl.kernel, mesh=mesh,
        out_type=jax.ShapeDtypeStruct((T, H), jnp.float32),
        scratch_types=[
            pltpu.VMEM((C_PER_W,), jnp.int32),
            pltpu.VMEM((C_PER_W,), jnp.int32),
            pltpu.VMEM((C_PER_W, H), jnp.float32),
            pltpu.VMEM((C_PER_W, H), jnp.float32),
            pltpu.VMEM((C_PER_W, 16), jnp.float32),
            pltpu.VMEM((C_PER_W, 16), jnp.float32),
            pltpu.SemaphoreType.DMA,
            pltpu.SemaphoreType.DMA,
        ])(_sc_combine_body)
    return f(sorted_out, p0, p1, w0r, w1r)


def _tc_body(be_ref, act_ref, x_ref, wgu_ref, wd_ref, acat_ref, gb_ref,
             ub_ref, da_ref, db_ref, out_ref):
    del be_ref
    b = pl.program_id(0)

    @pl.when(act_ref[b] == 1)
    def _compute():
        xb = x_ref[...]
        gu = jnp.dot(xb, wgu_ref[0], preferred_element_type=jnp.float32)
        xa = jnp.dot(xb, acat_ref[0], preferred_element_type=jnp.float32)
        lg = jnp.dot(xa[:, :R], gb_ref[0], preferred_element_type=jnp.float32)
        lu = jnp.dot(xa[:, R:], ub_ref[0], preferred_element_type=jnp.float32)
        gate = gu[:, :I] + lg
        up = gu[:, I:] + lu
        act = gate * jax.nn.sigmoid(gate) * up
        dn = jnp.dot(act, wd_ref[0], preferred_element_type=jnp.float32)
        x2 = jnp.dot(act, da_ref[0], preferred_element_type=jnp.float32)
        ld = jnp.dot(x2, db_ref[0], preferred_element_type=jnp.float32)
        out_ref[...] = dn + ld

    @pl.when(act_ref[b] == 0)
    def _pad():
        out_ref[...] = jnp.zeros_like(out_ref)


def kernel(hidden_states, topk_ids, topk_weights, gate_a, gate_b, up_a, up_b,
           down_a, down_b, weight_indices, seq_lens, lora_ranks, scalings,
           base_gate_up_weight, base_down_weight):
    del seq_lens, lora_ranks
    x = hidden_states.astype(jnp.float32)
    adapter = weight_indices[0]
    scaling = scalings[adapter].astype(jnp.float32)
    acat = jnp.transpose(
        jnp.concatenate([gate_a[adapter], up_a[adapter]], axis=1),
        (0, 2, 1)).astype(jnp.float32)
    gb = jnp.transpose(gate_b[adapter], (0, 2, 1)).astype(jnp.float32) * scaling
    ub = jnp.transpose(up_b[adapter], (0, 2, 1)).astype(jnp.float32) * scaling
    da = jnp.transpose(down_a[adapter], (0, 2, 1)).astype(jnp.float32)
    db = jnp.transpose(down_b[adapter], (0, 2, 1)).astype(jnp.float32) * scaling

    flat_ids = topk_ids.reshape(-1)
    flat_w = topk_weights.reshape(-1).astype(jnp.float32)
    oh = (flat_ids[:, None] == jnp.arange(E, dtype=jnp.int32)[None, :])
    g = jnp.sum(oh, axis=0, dtype=jnp.int32)
    blocks_e = (g + BLK - 1) // BLK
    pad_start = jnp.concatenate([jnp.zeros((1,), jnp.int32),
                                 jnp.cumsum(blocks_e)[:-1].astype(jnp.int32)]) * BLK
    rank = jnp.take_along_axis(jnp.cumsum(oh, axis=0, dtype=jnp.int32) - 1,
                               flat_ids[:, None].astype(jnp.int32), 1)[:, 0]
    dest = pad_start[flat_ids] + rank
    cumb = jnp.cumsum(blocks_e)
    block_expert = jnp.minimum(
        jnp.sum(jnp.arange(NB, dtype=jnp.int32)[:, None] >= cumb[None, :],
                axis=1), E - 1).astype(jnp.int32)
    block_active = (jnp.arange(NB, dtype=jnp.int32) < cumb[-1]).astype(jnp.int32)
    pos = dest.reshape(T, K)
    p0, p1 = pos[:, 0], pos[:, 1]
    w0r = jnp.broadcast_to(topk_weights[:, 0:1].astype(jnp.float32), (T, 16))
    w1r = jnp.broadcast_to(topk_weights[:, 1:2].astype(jnp.float32), (T, 16))

    sorted_x = _sc_distribute(x, p0, p1)

    grid_spec = pltpu.PrefetchScalarGridSpec(
        num_scalar_prefetch=2,
        grid=(NB,),
        in_specs=[
            pl.BlockSpec((BLK, H), lambda b, be, ac: (b, 0)),
            pl.BlockSpec((1, H, 2 * I), lambda b, be, ac: (be[b], 0, 0)),
            pl.BlockSpec((1, I, H), lambda b, be, ac: (be[b], 0, 0)),
            pl.BlockSpec((1, H, 2 * R), lambda b, be, ac: (be[b], 0, 0)),
            pl.BlockSpec((1, R, I), lambda b, be, ac: (be[b], 0, 0)),
            pl.BlockSpec((1, R, I), lambda b, be, ac: (be[b], 0, 0)),
            pl.BlockSpec((1, I, R), lambda b, be, ac: (be[b], 0, 0)),
            pl.BlockSpec((1, R, H), lambda b, be, ac: (be[b], 0, 0)),
        ],
        out_specs=pl.BlockSpec((BLK, H), lambda b, be, ac: (b, 0)),
    )
    sorted_out = pl.pallas_call(
        _tc_body,
        grid_spec=grid_spec,
        out_shape=jax.ShapeDtypeStruct((NPAD, H), jnp.float32),
        compiler_params=pltpu.CompilerParams(
            vmem_limit_bytes=100 * 1024 * 1024),
    )(block_expert, block_active, sorted_x,
      base_gate_up_weight, base_down_weight, acat, gb, ub, da, db)

    out = _sc_combine(sorted_out, p0, p1, w0r, w1r)
    return out.astype(hidden_states.dtype)
